# Initial kernel scaffold; baseline (speedup 1.0000x reference)
import jax
import jax.numpy as jnp
from jax.experimental import pallas as pl


def kernel(x, edge_index, edge_attr, emb, l0_W1, l0_b1, l0_W2, l0_b2, l0_eps, l1_W1, l1_b1, l1_W2, l1_b2, l1_eps, l2_W1, l2_b1, l2_W2, l2_b2, l2_eps, f_W1, f_b1, f_W2, f_b2):
    raise NotImplementedError("write your pallas kernel here")


# speedup vs baseline: 3.9071x; 3.9071x over previous
import functools

import jax
import jax.numpy as jnp
from jax import lax
from jax.experimental import pallas as pl
from jax.experimental.pallas import tpu as pltpu
from jax.experimental.pallas import tpu_sc as plsc

N = 10000
E = 320000
D = 128

NC = 2
NS = 16
NW = NC * NS
L = 16

DH = D // 2
EPS_ = E // NS
CH = 80
NCHUNK = EPS_ // CH

GB = 320
GPAD = NW * GB
GCH = 80
GNCH = GB // GCH

NP = 10240
RPS = NP // NS
ZR = 128


def _mesh():
    return plsc.VectorSubcoreMesh(core_axis_name="c", subcore_axis_name="s")


_SPLAT_DNUMS = lax.GatherDimensionNumbers(
    offset_dims=(), collapsed_slice_dims=(0,), start_index_map=(0,))


def _lane_splat(vec, j):
    return lax.gather(vec, jnp.full((L, 1), j, jnp.int32), _SPLAT_DNUMS, (1,),
                      mode=lax.GatherScatterMode.PROMISE_IN_BOUNDS)


def _emb_gather(emb, idx_pad):
    @functools.partial(
        pl.kernel,
        out_type=jax.ShapeDtypeStruct((GPAD, D), jnp.float32),
        mesh=_mesh(),
        scratch_types=[
            pltpu.VMEM((GNCH, GCH), jnp.int32),
            pltpu.VMEM((GCH, D), jnp.float32),
            pltpu.SemaphoreType.DMA,
        ],
    )
    def k(emb_hbm, idx_hbm, out_hbm, idx_v, rows_v, sem):
        wid = lax.axis_index("s") * NC + lax.axis_index("c")
        pltpu.sync_copy(idx_hbm.at[wid], idx_v)

        def body(j, carry):
            pltpu.async_copy(emb_hbm.at[idx_v.at[j]], rows_v, sem).wait()
            pltpu.sync_copy(rows_v, out_hbm.at[pl.ds(wid * GB + j * GCH, GCH)])
            return carry

        lax.fori_loop(0, GNCH, body, 0)

    return k(emb, idx_pad)


def _seg_partials(hr, srcb, dstb, eab):
    @functools.partial(
        pl.kernel,
        out_type=jax.ShapeDtypeStruct((NC * NP, DH), jnp.float32),
        mesh=_mesh(),
        compiler_params=pltpu.CompilerParams(use_tc_tiling_on_sc=False),
        scratch_types=[
            pltpu.VMEM((NCHUNK, CH), jnp.int32),
            pltpu.VMEM((NCHUNK, CH), jnp.int32),
            pltpu.VMEM((NCHUNK, CH), jnp.float32),
            pltpu.VMEM((CH,), jnp.int32),
            pltpu.VMEM((CH, DH), jnp.float32),
            pltpu.VMEM((ZR, DH), jnp.float32),
            pltpu.VMEM_SHARED((NP, DH), jnp.float32),
            pltpu.SemaphoreType.DMA,
        ],
    )
    def k(hr_hbm, src_hbm, dst_hbm, ea_hbm, out_hbm,
          src_v, dst_v, ea_v, idx2_v, rows_v, z_v, acc_sh, sem):
        c = lax.axis_index("c")
        s = lax.axis_index("s")

        zero16 = jnp.zeros((L,), jnp.float32)

        def zfill(i, carry):
            for kk in range(DH // L):
                z_v[i, pl.ds(kk * L, L)] = zero16
            return carry

        lax.fori_loop(0, ZR, zfill, 0)
        for q in range(RPS // ZR):
            pltpu.sync_copy(z_v, acc_sh.at[pl.ds(s * RPS + q * ZR, ZR)])
        plsc.subcore_barrier()

        pltpu.sync_copy(src_hbm.at[s], src_v)
        pltpu.sync_copy(dst_hbm.at[s], dst_v)
        pltpu.sync_copy(ea_hbm.at[s], ea_v)

        def chunk(jc, carry):
            for g in range(CH // L):
                sl = pl.ds(g * L, L)
                idx2_v[sl] = src_v[jc, sl] * 2 + c
            pltpu.async_copy(hr_hbm.at[idx2_v], rows_v, sem).wait()

            def group(g, inner):
                ea16 = ea_v[jc, pl.ds(g * L, L)]
                for j in range(L):
                    w = _lane_splat(ea16, j)
                    r = g * L + j
                    for kk in range(DH // L):
                        sl = pl.ds(kk * L, L)
                        rows_v[r, sl] = rows_v[r, sl] * w
                return inner

            lax.fori_loop(0, CH // L, group, 0)
            pltpu.sync_copy(rows_v, acc_sh.at[dst_v.at[jc]], add=True)
            return carry

        lax.fori_loop(0, NCHUNK, chunk, 0)
        plsc.subcore_barrier()

        for q in range(RPS // ZR):
            base = s * RPS + q * ZR
            pltpu.sync_copy(acc_sh.at[pl.ds(base, ZR)], z_v)
            pltpu.sync_copy(z_v, out_hbm.at[pl.ds(c * NP + base, ZR)])

    return k(hr, srcb, dstb, eab)


def _mlp_layer(h, p0, p1, scale, W1, b1, W2, b2):
    BM = 1000

    def body(s_ref, h_ref, p0_ref, p1_ref, W1_ref, b1_ref, W2_ref, b2_ref,
             o_ref):
        agg = jnp.concatenate([p0_ref[...], p1_ref[...]], axis=1)
        hb = h_ref[...] * s_ref[0, 0] + agg
        a = jnp.dot(hb, W1_ref[...], preferred_element_type=jnp.float32)
        a = jnp.maximum(a + b1_ref[...], 0.0)
        o = jnp.dot(a, W2_ref[...], preferred_element_type=jnp.float32)
        o_ref[...] = jnp.maximum(o + b2_ref[...], 0.0)

    return pl.pallas_call(
        body,
        grid=(N // BM,),
        in_specs=[
            pl.BlockSpec(memory_space=pltpu.SMEM),
            pl.BlockSpec((BM, D), lambda i: (i, 0)),
            pl.BlockSpec((BM, DH), lambda i: (i, 0)),
            pl.BlockSpec((BM, DH), lambda i: (i, 0)),
            pl.BlockSpec((D, D), lambda i: (0, 0)),
            pl.BlockSpec((1, D), lambda i: (0, 0)),
            pl.BlockSpec((D, D), lambda i: (0, 0)),
            pl.BlockSpec((1, D), lambda i: (0, 0)),
        ],
        out_specs=pl.BlockSpec((BM, D), lambda i: (i, 0)),
        out_shape=jax.ShapeDtypeStruct((N, D), jnp.float32),
    )(scale, h, p0, p1, W1, b1, W2, b2)


def _head(h, W1p, b1p, W2p, b2p):
    def body(h_ref, w1_ref, b1_ref, w2_ref, b2_ref, o_ref):
        g = jnp.mean(h_ref[...], axis=0, keepdims=True)
        z = jnp.dot(g, w1_ref[...], preferred_element_type=jnp.float32)
        z = jnp.maximum(z + b1_ref[...], 0.0)
        o = jnp.dot(z, w2_ref[...], preferred_element_type=jnp.float32)
        o = o + b2_ref[...]
        col = lax.broadcasted_iota(jnp.int32, (1, D), 1)
        o = jnp.where(col < 2, o, -1e30)
        m = jnp.max(o, axis=1, keepdims=True)
        e = jnp.exp(o - m)
        o_ref[...] = e / jnp.sum(e, axis=1, keepdims=True)

    return pl.pallas_call(
        body,
        out_shape=jax.ShapeDtypeStruct((1, D), jnp.float32),
    )(h, W1p, b1p, W2p, b2p)


def kernel(x, edge_index, edge_attr, emb,
           l0_W1, l0_b1, l0_W2, l0_b2, l0_eps,
           l1_W1, l1_b1, l1_W2, l1_b2, l1_eps,
           l2_W1, l2_b1, l2_W2, l2_b2, l2_eps,
           f_W1, f_b1, f_W2, f_b2):
    x_idx = x.reshape(-1).astype(jnp.int32)
    xp = jnp.concatenate(
        [x_idx, jnp.zeros((GPAD - N,), jnp.int32)]).reshape(NW, GNCH, GCH)
    src = edge_index[0].astype(jnp.int32).reshape(NS, NCHUNK, CH)
    dst = edge_index[1].astype(jnp.int32).reshape(NS, NCHUNK, CH)
    eab = edge_attr.astype(jnp.float32).reshape(NS, NCHUNK, CH)

    h = _emb_gather(emb, xp)[:N]

    layers = [(l0_W1, l0_b1, l0_W2, l0_b2, l0_eps),
              (l1_W1, l1_b1, l1_W2, l1_b2, l1_eps),
              (l2_W1, l2_b1, l2_W2, l2_b2, l2_eps)]
    for (W1, b1, W2, b2, eps) in layers:
        parts = _seg_partials(h.reshape(2 * N, DH), src, dst, eab)
        scale = (2.0 + eps).reshape(1, 1)
        h = _mlp_layer(h, parts[:N], parts[NP:NP + N], scale,
                       W1, b1.reshape(1, D), W2, b2.reshape(1, D))

    W1p = jnp.pad(f_W1, ((0, 0), (0, D - f_W1.shape[1])))
    b1p = jnp.pad(f_b1, (0, D - f_b1.shape[0])).reshape(1, D)
    W2p = jnp.pad(f_W2, ((0, D - f_W2.shape[0]), (0, D - f_W2.shape[1])))
    b2p = jnp.pad(f_b2, (0, D - f_b2.shape[0])).reshape(1, D)
    out = _head(h, W1p, b1p, W2p, b2p)
    return out[:, :2]


# speedup vs baseline: 4.8528x; 1.2420x over previous
import functools

import jax
import jax.numpy as jnp
from jax import lax
from jax.experimental import pallas as pl
from jax.experimental.pallas import tpu as pltpu
from jax.experimental.pallas import tpu_sc as plsc

N = 10000
E = 320000
D = 128

NC = 2
NS = 16
NW = NC * NS
L = 16

DH = D // 2
CH = 128
NCHUNK = 158
EPAD = NS * NCHUNK * CH

GB = 320
GPAD = NW * GB
GCH = 80
GNCH = GB // GCH

NP = 10240
RPS = NP // NS
ZR = 128


def _mesh():
    return plsc.VectorSubcoreMesh(core_axis_name="c", subcore_axis_name="s")


_SPLAT_DNUMS = lax.GatherDimensionNumbers(
    offset_dims=(), collapsed_slice_dims=(0,), start_index_map=(0,))


def _lane_splat(vec, j):
    return lax.gather(vec, jnp.full((L, 1), j, jnp.int32), _SPLAT_DNUMS, (1,),
                      mode=lax.GatherScatterMode.PROMISE_IN_BOUNDS)


def _emb_gather(emb, idx_pad):
    @functools.partial(
        pl.kernel,
        out_type=jax.ShapeDtypeStruct((GPAD, D), jnp.float32),
        mesh=_mesh(),
        scratch_types=[
            pltpu.VMEM((GNCH, GCH), jnp.int32),
            pltpu.VMEM((GCH, D), jnp.float32),
            pltpu.SemaphoreType.DMA,
        ],
    )
    def k(emb_hbm, idx_hbm, out_hbm, idx_v, rows_v, sem):
        wid = lax.axis_index("s") * NC + lax.axis_index("c")
        pltpu.sync_copy(idx_hbm.at[wid], idx_v)

        def body(j, carry):
            pltpu.async_copy(emb_hbm.at[idx_v.at[j]], rows_v, sem).wait()
            pltpu.sync_copy(rows_v, out_hbm.at[pl.ds(wid * GB + j * GCH, GCH)])
            return carry

        lax.fori_loop(0, GNCH, body, 0)

    return k(emb, idx_pad)


def _seg_partials(hr, srcb, dstb, eab):
    @functools.partial(
        pl.kernel,
        out_type=jax.ShapeDtypeStruct((NC * NP, DH), jnp.float32),
        mesh=_mesh(),
        compiler_params=pltpu.CompilerParams(use_tc_tiling_on_sc=False),
        scratch_types=[
            pltpu.VMEM((NCHUNK, CH), jnp.int32),
            pltpu.VMEM((NCHUNK, CH), jnp.int32),
            pltpu.VMEM((NCHUNK, CH), jnp.float32),
            pltpu.VMEM((CH,), jnp.int32),
            pltpu.VMEM((CH,), jnp.int32),
            pltpu.VMEM((CH, DH), jnp.float32),
            pltpu.VMEM((CH, DH), jnp.float32),
            pltpu.VMEM((ZR, DH), jnp.float32),
            pltpu.VMEM_SHARED((NP, DH), jnp.float32),
            pltpu.SemaphoreType.DMA,
            pltpu.SemaphoreType.DMA,
            pltpu.SemaphoreType.DMA,
            pltpu.SemaphoreType.DMA,
        ],
    )
    def k(hr_hbm, src_hbm, dst_hbm, ea_hbm, out_hbm,
          src_v, dst_v, ea_v, idx_0, idx_1, rows_0, rows_1, z_v, acc_sh,
          sg_0, sg_1, ss_0, ss_1):
        c = lax.axis_index("c")
        s = lax.axis_index("s")
        idx = (idx_0, idx_1)
        rows = (rows_0, rows_1)
        sg = (sg_0, sg_1)
        ss = (ss_0, ss_1)

        zero16 = jnp.zeros((L,), jnp.float32)

        def zfill(i, carry):
            for kk in range(DH // L):
                z_v[i, pl.ds(kk * L, L)] = zero16
            return carry

        lax.fori_loop(0, ZR, zfill, 0)
        for q in range(RPS // ZR):
            pltpu.sync_copy(z_v, acc_sh.at[pl.ds(s * RPS + q * ZR, ZR)])
        plsc.subcore_barrier()

        pltpu.sync_copy(src_hbm.at[s], src_v)
        pltpu.sync_copy(dst_hbm.at[s], dst_v)
        pltpu.sync_copy(ea_hbm.at[s], ea_v)

        def make_idx(b, jc):
            def g_(g, carry):
                sl = pl.ds(g * L, L)
                idx[b][sl] = src_v[jc, sl] * 2 + c
                return carry
            lax.fori_loop(0, CH // L, g_, 0)

        def gather_start(b):
            pltpu.async_copy(hr_hbm.at[idx[b]], rows[b], sg[b])

        def gather_wait(b):
            pltpu.make_async_copy(hr_hbm.at[idx[b]], rows[b], sg[b]).wait()

        def scale(b, jc):
            def group(g, inner):
                ea16 = ea_v[jc, pl.ds(g * L, L)]
                for j in range(L):
                    w = _lane_splat(ea16, j)
                    r = g * L + j
                    for kk in range(DH // L):
                        sl = pl.ds(kk * L, L)
                        rows[b][r, sl] = rows[b][r, sl] * w
                return inner
            lax.fori_loop(0, CH // L, group, 0)

        def scatter_start(b, jc):
            pltpu.async_copy(rows[b], acc_sh.at[dst_v.at[jc]], ss[b],
                             add=True)

        def scatter_wait(b, jc):
            pltpu.make_async_copy(rows[b], acc_sh.at[dst_v.at[jc]],
                                  ss[b]).wait()

        make_idx(0, 0)
        gather_start(0)
        make_idx(1, 1)
        gather_start(1)
        gather_wait(0)
        scale(0, 0)
        scatter_start(0, 0)
        make_idx(0, 2)

        def pair(p, carry):
            j1 = 2 * p + 1
            scatter_wait(0, j1 - 1)
            gather_start(0)
            gather_wait(1)
            scale(1, j1)
            scatter_start(1, j1)
            make_idx(1, jnp.minimum(j1 + 2, NCHUNK - 1))
            scatter_wait(1, j1)
            gather_start(1)
            gather_wait(0)
            scale(0, j1 + 1)
            scatter_start(0, j1 + 1)
            make_idx(0, jnp.minimum(j1 + 3, NCHUNK - 1))
            return carry

        lax.fori_loop(0, (NCHUNK - 2) // 2, pair, 0)
        gather_wait(1)
        scale(1, NCHUNK - 1)
        scatter_start(1, NCHUNK - 1)
        scatter_wait(0, NCHUNK - 2)
        scatter_wait(1, NCHUNK - 1)
        plsc.subcore_barrier()

        for q in range(RPS // ZR):
            base = s * RPS + q * ZR
            pltpu.sync_copy(acc_sh.at[pl.ds(base, ZR)], z_v)
            pltpu.sync_copy(z_v, out_hbm.at[pl.ds(c * NP + base, ZR)])

    return k(hr, srcb, dstb, eab)


def _mlp_layer(h, p0, p1, scale, W1, b1, W2, b2):
    BM = 1000

    def body(s_ref, h_ref, p0_ref, p1_ref, W1_ref, b1_ref, W2_ref, b2_ref,
             o_ref):
        agg = jnp.concatenate([p0_ref[...], p1_ref[...]], axis=1)
        hb = h_ref[...] * s_ref[0, 0] + agg
        a = jnp.dot(hb, W1_ref[...], preferred_element_type=jnp.float32)
        a = jnp.maximum(a + b1_ref[...], 0.0)
        o = jnp.dot(a, W2_ref[...], preferred_element_type=jnp.float32)
        o_ref[...] = jnp.maximum(o + b2_ref[...], 0.0)

    return pl.pallas_call(
        body,
        grid=(N // BM,),
        in_specs=[
            pl.BlockSpec(memory_space=pltpu.SMEM),
            pl.BlockSpec((BM, D), lambda i: (i, 0)),
            pl.BlockSpec((BM, DH), lambda i: (i, 0)),
            pl.BlockSpec((BM, DH), lambda i: (i, 0)),
            pl.BlockSpec((D, D), lambda i: (0, 0)),
            pl.BlockSpec((1, D), lambda i: (0, 0)),
            pl.BlockSpec((D, D), lambda i: (0, 0)),
            pl.BlockSpec((1, D), lambda i: (0, 0)),
        ],
        out_specs=pl.BlockSpec((BM, D), lambda i: (i, 0)),
        out_shape=jax.ShapeDtypeStruct((N, D), jnp.float32),
    )(scale, h, p0, p1, W1, b1, W2, b2)


def _head(h, W1p, b1p, W2p, b2p):
    def body(h_ref, w1_ref, b1_ref, w2_ref, b2_ref, o_ref):
        g = jnp.mean(h_ref[...], axis=0, keepdims=True)
        z = jnp.dot(g, w1_ref[...], preferred_element_type=jnp.float32)
        z = jnp.maximum(z + b1_ref[...], 0.0)
        o = jnp.dot(z, w2_ref[...], preferred_element_type=jnp.float32)
        o = o + b2_ref[...]
        col = lax.broadcasted_iota(jnp.int32, (1, D), 1)
        o = jnp.where(col < 2, o, -1e30)
        m = jnp.max(o, axis=1, keepdims=True)
        e = jnp.exp(o - m)
        o_ref[...] = e / jnp.sum(e, axis=1, keepdims=True)

    return pl.pallas_call(
        body,
        out_shape=jax.ShapeDtypeStruct((1, D), jnp.float32),
    )(h, W1p, b1p, W2p, b2p)


def kernel(x, edge_index, edge_attr, emb,
           l0_W1, l0_b1, l0_W2, l0_b2, l0_eps,
           l1_W1, l1_b1, l1_W2, l1_b2, l1_eps,
           l2_W1, l2_b1, l2_W2, l2_b2, l2_eps,
           f_W1, f_b1, f_W2, f_b2):
    x_idx = x.reshape(-1).astype(jnp.int32)
    xp = jnp.concatenate(
        [x_idx, jnp.zeros((GPAD - N,), jnp.int32)]).reshape(NW, GNCH, GCH)
    zpad = jnp.zeros((EPAD - E,), jnp.int32)
    src = jnp.concatenate(
        [edge_index[0].astype(jnp.int32), zpad]).reshape(NS, NCHUNK, CH)
    dst = jnp.concatenate(
        [edge_index[1].astype(jnp.int32), zpad]).reshape(NS, NCHUNK, CH)
    eab = jnp.concatenate(
        [edge_attr.astype(jnp.float32),
         jnp.zeros((EPAD - E,), jnp.float32)]).reshape(NS, NCHUNK, CH)

    h = _emb_gather(emb, xp)[:N]

    layers = [(l0_W1, l0_b1, l0_W2, l0_b2, l0_eps),
              (l1_W1, l1_b1, l1_W2, l1_b2, l1_eps),
              (l2_W1, l2_b1, l2_W2, l2_b2, l2_eps)]
    for (W1, b1, W2, b2, eps) in layers:
        parts = _seg_partials(h.reshape(2 * N, DH), src, dst, eab)
        scale = (2.0 + eps).reshape(1, 1)
        h = _mlp_layer(h, parts[:N], parts[NP:NP + N], scale,
                       W1, b1.reshape(1, D), W2, b2.reshape(1, D))

    W1p = jnp.pad(f_W1, ((0, 0), (0, D - f_W1.shape[1])))
    b1p = jnp.pad(f_b1, (0, D - f_b1.shape[0])).reshape(1, D)
    W2p = jnp.pad(f_W2, ((0, D - f_W2.shape[0]), (0, D - f_W2.shape[1])))
    b2p = jnp.pad(f_b2, (0, D - f_b2.shape[0])).reshape(1, D)
    out = _head(h, W1p, b1p, W2p, b2p)
    return out[:, :2]


# speedup vs baseline: 5.8820x; 1.2121x over previous
import functools

import jax
import jax.numpy as jnp
from jax import lax
from jax.experimental import pallas as pl
from jax.experimental.pallas import tpu as pltpu
from jax.experimental.pallas import tpu_sc as plsc

N = 10000
E = 320000
D = 128

NC = 2
NS = 16
NW = NC * NS
L = 16

DH = D // 2
CH = 128
NCHUNK = 160
NBLK = 32
EPAD = NS * NCHUNK * CH

GB = 320
GPAD = NW * GB
GCH = 80
GNCH = GB // GCH

NP = 10240
RPS = NP // NS
ZR = 128


def _mesh():
    return plsc.VectorSubcoreMesh(core_axis_name="c", subcore_axis_name="s")


_SPLAT_DNUMS = lax.GatherDimensionNumbers(
    offset_dims=(), collapsed_slice_dims=(0,), start_index_map=(0,))


def _lane_splat(vec, j):
    return lax.gather(vec, jnp.full((L, 1), j, jnp.int32), _SPLAT_DNUMS, (1,),
                      mode=lax.GatherScatterMode.PROMISE_IN_BOUNDS)


def _emb_gather(emb, idx_pad):
    @functools.partial(
        pl.kernel,
        out_type=jax.ShapeDtypeStruct((GPAD, D), jnp.float32),
        mesh=_mesh(),
        scratch_types=[
            pltpu.VMEM((GNCH, GCH), jnp.int32),
            pltpu.VMEM((GCH, D), jnp.float32),
            pltpu.SemaphoreType.DMA,
        ],
    )
    def k(emb_hbm, idx_hbm, out_hbm, idx_v, rows_v, sem):
        wid = lax.axis_index("s") * NC + lax.axis_index("c")
        pltpu.sync_copy(idx_hbm.at[wid], idx_v)

        def body(j, carry):
            pltpu.async_copy(emb_hbm.at[idx_v.at[j]], rows_v, sem).wait()
            pltpu.sync_copy(rows_v, out_hbm.at[pl.ds(wid * GB + j * GCH, GCH)])
            return carry

        lax.fori_loop(0, GNCH, body, 0)

    return k(emb, idx_pad)


def _seg_partials(hr, srcb, dstb, eab):
    @functools.partial(
        pl.kernel,
        out_type=jax.ShapeDtypeStruct((NC * NP, DH), jnp.float32),
        mesh=_mesh(),
        compiler_params=pltpu.CompilerParams(use_tc_tiling_on_sc=False),
        scratch_types=[
            pltpu.VMEM((2 * NBLK, CH), jnp.int32),
            pltpu.VMEM((2 * NBLK, CH), jnp.int32),
            pltpu.VMEM((2 * NBLK, CH), jnp.float32),
            pltpu.VMEM((CH,), jnp.int32),
            pltpu.VMEM((CH,), jnp.int32),
            pltpu.VMEM((CH,), jnp.int32),
            pltpu.VMEM((CH,), jnp.int32),
            pltpu.VMEM((CH, DH), jnp.float32),
            pltpu.VMEM((CH, DH), jnp.float32),
            pltpu.VMEM((CH, DH), jnp.float32),
            pltpu.VMEM((CH, DH), jnp.float32),
            pltpu.VMEM_SHARED((NP, DH), jnp.float32),
            pltpu.SemaphoreType.DMA,
            pltpu.SemaphoreType.DMA,
            pltpu.SemaphoreType.DMA,
            pltpu.SemaphoreType.DMA,
            pltpu.SemaphoreType.DMA,
            pltpu.SemaphoreType.DMA,
            pltpu.SemaphoreType.DMA,
            pltpu.SemaphoreType.DMA,
        ],
    )
    def k(hr_hbm, src_hbm, dst_hbm, ea_hbm, out_hbm,
          src_v, dst_v, ea_v, idx_0, idx_1, idx_2, idx_3,
          rows_0, rows_1, rows_2, rows_3, acc_sh,
          sg_0, sg_1, sg_2, sg_3, ss_0, ss_1, ss_2, ss_3):
        c = lax.axis_index("c")
        s = lax.axis_index("s")
        idx = (idx_0, idx_1, idx_2, idx_3)
        rows = (rows_0, rows_1, rows_2, rows_3)
        sg = (sg_0, sg_1, sg_2, sg_3)
        ss = (ss_0, ss_1, ss_2, ss_3)

        zero16 = jnp.zeros((L,), jnp.float32)

        def zfill(i, carry):
            for kk in range(DH // L):
                rows_0[i, pl.ds(kk * L, L)] = zero16
            return carry

        lax.fori_loop(0, ZR, zfill, 0)
        for q in range(RPS // ZR):
            pltpu.sync_copy(rows_0, acc_sh.at[pl.ds(s * RPS + q * ZR, ZR)])
        plsc.subcore_barrier()

        def stage_block(blk):
            slot = lax.rem(blk, 2)
            hrow = blk * NBLK
            vrow = slot * NBLK
            pltpu.sync_copy(src_hbm.at[s, pl.ds(hrow, NBLK)],
                            src_v.at[pl.ds(vrow, NBLK)])
            pltpu.sync_copy(dst_hbm.at[s, pl.ds(hrow, NBLK)],
                            dst_v.at[pl.ds(vrow, NBLK)])
            pltpu.sync_copy(ea_hbm.at[s, pl.ds(hrow, NBLK)],
                            ea_v.at[pl.ds(vrow, NBLK)])

        def erow(j):
            return ((j // NBLK) % 2) * NBLK + j % NBLK

        stage_block(0)
        stage_block(1)

        def make_idx(b, jc):
            jr = erow(jc)
            def g_(g, carry):
                sl = pl.ds(g * L, L)
                idx[b][sl] = src_v[jr, sl] * 2 + c
                return carry
            lax.fori_loop(0, CH // L, g_, 0)

        def gather_start(b):
            pltpu.async_copy(hr_hbm.at[idx[b]], rows[b], sg[b])

        def gather_wait(b):
            pltpu.make_async_copy(hr_hbm.at[idx[b]], rows[b], sg[b]).wait()

        def scale(b, jc):
            jr = erow(jc)
            @functools.partial(plsc.parallel_loop, 0, CH // L, unroll=2)
            def group(g):
                ea16 = ea_v[jr, pl.ds(g * L, L)]
                for j in range(L):
                    w = _lane_splat(ea16, j)
                    r = g * L + j
                    for kk in range(DH // L):
                        sl = pl.ds(kk * L, L)
                        rows[b][r, sl] = rows[b][r, sl] * w

        def scatter_start(b, jc):
            pltpu.async_copy(rows[b], acc_sh.at[dst_v.at[erow(jc)]], ss[b],
                             add=True)

        def scatter_wait(b, jc):
            pltpu.make_async_copy(rows[b], acc_sh.at[dst_v.at[erow(jc)]],
                                  ss[b]).wait()

        make_idx(0, 0)
        gather_start(0)
        make_idx(1, 1)
        gather_start(1)
        for j0 in (0, 1):
            make_idx(j0 + 2, j0 + 2)
            gather_start(j0 + 2)
            gather_wait(j0)
            scale(j0, j0)
            scatter_start(j0, j0)

        def quad(q, carry):
            jb = 4 * q + 2
            @pl.when(jnp.logical_and(lax.rem(q, 8) == 0,
                                     jnp.logical_and(q > 0, q < 32)))
            def _():
                stage_block(q // 8 + 1)
            for t in range(4):
                b = (2 + t) % 4
                f = t
                j = jb + t
                scatter_wait(f, j - 2)
                make_idx(f, j + 2)
                gather_start(f)
                gather_wait(b)
                scale(b, j)
                scatter_start(b, j)
            return carry

        lax.fori_loop(0, (NCHUNK - 4) // 4, quad, 0)
        for t, b in ((2, 2), (1, 3)):
            j = NCHUNK - t
            scatter_wait(b - 2, j - 2)
            gather_wait(b)
            scale(b, j)
            scatter_start(b, j)
        scatter_wait(2, NCHUNK - 2)
        scatter_wait(3, NCHUNK - 1)
        plsc.subcore_barrier()

        for q in range(RPS // ZR):
            base = s * RPS + q * ZR
            pltpu.sync_copy(acc_sh.at[pl.ds(base, ZR)], rows_0)
            pltpu.sync_copy(rows_0, out_hbm.at[pl.ds(c * NP + base, ZR)])

    return k(hr, srcb, dstb, eab)


def _mlp_layer(h, p0, p1, scale, W1, b1, W2, b2):
    BM = 1000

    def body(s_ref, h_ref, p0_ref, p1_ref, W1_ref, b1_ref, W2_ref, b2_ref,
             o_ref):
        agg = jnp.concatenate([p0_ref[...], p1_ref[...]], axis=1)
        hb = h_ref[...] * s_ref[0, 0] + agg
        a = jnp.dot(hb, W1_ref[...], preferred_element_type=jnp.float32)
        a = jnp.maximum(a + b1_ref[...], 0.0)
        o = jnp.dot(a, W2_ref[...], preferred_element_type=jnp.float32)
        o_ref[...] = jnp.maximum(o + b2_ref[...], 0.0)

    return pl.pallas_call(
        body,
        grid=(N // BM,),
        in_specs=[
            pl.BlockSpec(memory_space=pltpu.SMEM),
            pl.BlockSpec((BM, D), lambda i: (i, 0)),
            pl.BlockSpec((BM, DH), lambda i: (i, 0)),
            pl.BlockSpec((BM, DH), lambda i: (i, 0)),
            pl.BlockSpec((D, D), lambda i: (0, 0)),
            pl.BlockSpec((1, D), lambda i: (0, 0)),
            pl.BlockSpec((D, D), lambda i: (0, 0)),
            pl.BlockSpec((1, D), lambda i: (0, 0)),
        ],
        out_specs=pl.BlockSpec((BM, D), lambda i: (i, 0)),
        out_shape=jax.ShapeDtypeStruct((N, D), jnp.float32),
    )(scale, h, p0, p1, W1, b1, W2, b2)


def _head(h, W1p, b1p, W2p, b2p):
    def body(h_ref, w1_ref, b1_ref, w2_ref, b2_ref, o_ref):
        g = jnp.mean(h_ref[...], axis=0, keepdims=True)
        z = jnp.dot(g, w1_ref[...], preferred_element_type=jnp.float32)
        z = jnp.maximum(z + b1_ref[...], 0.0)
        o = jnp.dot(z, w2_ref[...], preferred_element_type=jnp.float32)
        o = o + b2_ref[...]
        col = lax.broadcasted_iota(jnp.int32, (1, D), 1)
        o = jnp.where(col < 2, o, -1e30)
        m = jnp.max(o, axis=1, keepdims=True)
        e = jnp.exp(o - m)
        o_ref[...] = e / jnp.sum(e, axis=1, keepdims=True)

    return pl.pallas_call(
        body,
        out_shape=jax.ShapeDtypeStruct((1, D), jnp.float32),
    )(h, W1p, b1p, W2p, b2p)


def kernel(x, edge_index, edge_attr, emb,
           l0_W1, l0_b1, l0_W2, l0_b2, l0_eps,
           l1_W1, l1_b1, l1_W2, l1_b2, l1_eps,
           l2_W1, l2_b1, l2_W2, l2_b2, l2_eps,
           f_W1, f_b1, f_W2, f_b2):
    x_idx = x.reshape(-1).astype(jnp.int32)
    xp = jnp.concatenate(
        [x_idx, jnp.zeros((GPAD - N,), jnp.int32)]).reshape(NW, GNCH, GCH)
    zpad = jnp.zeros((EPAD - E,), jnp.int32)
    src = jnp.concatenate(
        [edge_index[0].astype(jnp.int32), zpad]).reshape(NS, NCHUNK, CH)
    dst = jnp.concatenate(
        [edge_index[1].astype(jnp.int32), zpad]).reshape(NS, NCHUNK, CH)
    eab = jnp.concatenate(
        [edge_attr.astype(jnp.float32),
         jnp.zeros((EPAD - E,), jnp.float32)]).reshape(NS, NCHUNK, CH)

    h = _emb_gather(emb, xp)[:N]

    layers = [(l0_W1, l0_b1, l0_W2, l0_b2, l0_eps),
              (l1_W1, l1_b1, l1_W2, l1_b2, l1_eps),
              (l2_W1, l2_b1, l2_W2, l2_b2, l2_eps)]
    for (W1, b1, W2, b2, eps) in layers:
        parts = _seg_partials(h.reshape(2 * N, DH), src, dst, eab)
        scale = (2.0 + eps).reshape(1, 1)
        h = _mlp_layer(h, parts[:N], parts[NP:NP + N], scale,
                       W1, b1.reshape(1, D), W2, b2.reshape(1, D))

    W1p = jnp.pad(f_W1, ((0, 0), (0, D - f_W1.shape[1])))
    b1p = jnp.pad(f_b1, (0, D - f_b1.shape[0])).reshape(1, D)
    W2p = jnp.pad(f_W2, ((0, D - f_W2.shape[0]), (0, D - f_W2.shape[1])))
    b2p = jnp.pad(f_b2, (0, D - f_b2.shape[0])).reshape(1, D)
    out = _head(h, W1p, b1p, W2p, b2p)
    return out[:, :2]
